# Initial kernel scaffold; baseline (speedup 1.0000x reference)
import jax
import jax.numpy as jnp
from jax.experimental import pallas as pl


def kernel(x, table):
    raise NotImplementedError("write your pallas kernel here")


# speedup vs baseline: 1.6573x; 1.6573x over previous
import functools

import jax
import jax.numpy as jnp
from jax import lax
from jax.experimental import pallas as pl
from jax.experimental.pallas import tpu as pltpu
from jax.experimental.pallas import tpu_sc as plsc

VOCAB = 1000000
DIM = 32
B = 4096
L = 50
K = 4

NC = 2
NS = 16
NW = NC * NS
NTOK = B * L
TPW = NTOK // NW
T = 256
NCHUNK = TPW // T
RPC = T * K
GW = 128
G = RPC // GW
IDX_ROWS = NTOK * K // GW

_mesh = plsc.VectorSubcoreMesh(core_axis_name="c", subcore_axis_name="s")


@functools.partial(
    pl.kernel,
    mesh=_mesh,
    out_type=jax.ShapeDtypeStruct((NTOK, DIM), jnp.float32),
    compiler_params=pltpu.CompilerParams(use_tc_tiling_on_sc=False),
    scratch_types=[
        pltpu.VMEM((G, GW), jnp.int32),
        pltpu.VMEM((RPC, DIM), jnp.float32),
        pltpu.VMEM((T, DIM), jnp.float32),
        pltpu.SemaphoreType.DMA,
    ],
)
def _emb_lookup(table_hbm, idx_hbm, out_hbm, idx_v, rows_v, out_v, sem):
    wid = lax.axis_index("s") * NC + lax.axis_index("c")

    def chunk_body(c, carry):
        tok_base = wid * TPW + c * T
        idx_row_base = wid * (TPW * K // GW) + c * G
        pltpu.sync_copy(idx_hbm.at[pl.ds(idx_row_base, G)], idx_v)
        copies = [
            pltpu.async_copy(
                table_hbm.at[idx_v.at[g]],
                rows_v.at[pl.ds(g * GW, GW)],
                sem,
            )
            for g in range(G)
        ]
        for cp in copies:
            cp.wait()

        def tok_body(t, tcarry):
            r = t * K
            for h in range(DIM // 16):
                d = pl.ds(h * 16, 16)
                out_v[t, d] = (
                    rows_v[r, d]
                    + rows_v[r + 1, d]
                    + rows_v[r + 2, d]
                    + rows_v[r + 3, d]
                )
            return tcarry

        lax.fori_loop(0, T, tok_body, 0)
        pltpu.sync_copy(out_v, out_hbm.at[pl.ds(tok_base, T)])
        return carry

    lax.fori_loop(0, NCHUNK, chunk_body, 0)


def kernel(x, table):
    idx = x.reshape(NTOK * K).astype(jnp.int32).reshape(IDX_ROWS, GW)
    out = _emb_lookup(table, idx)
    return out.reshape(B, L, DIM)


# speedup vs baseline: 1.7795x; 1.0737x over previous
import functools

import jax
import jax.numpy as jnp
from jax import lax
from jax.experimental import pallas as pl
from jax.experimental.pallas import tpu as pltpu
from jax.experimental.pallas import tpu_sc as plsc

VOCAB = 1000000
DIM = 32
B = 4096
L = 50
K = 4

NC = 2
NS = 16
NW = NC * NS
NTOK = B * L
TPW = NTOK // NW
T = 256
NCHUNK = TPW // T
RPC = T * K
GW = 128
G = RPC // GW
IDX_ROWS = NTOK * K // GW
IDX_RPW = IDX_ROWS // NW
NBUF = 2

_mesh = plsc.VectorSubcoreMesh(core_axis_name="c", subcore_axis_name="s")


@functools.partial(
    pl.kernel,
    mesh=_mesh,
    out_type=jax.ShapeDtypeStruct((NTOK, DIM), jnp.float32),
    compiler_params=pltpu.CompilerParams(use_tc_tiling_on_sc=False),
    scratch_types=[
        pltpu.VMEM((IDX_RPW, GW), jnp.int32),
        pltpu.VMEM((NBUF, RPC, DIM), jnp.float32),
        pltpu.VMEM((NBUF, T, DIM), jnp.float32),
        pltpu.SemaphoreType.DMA,
        pltpu.SemaphoreType.DMA,
        pltpu.SemaphoreType.DMA,
        pltpu.SemaphoreType.DMA,
    ],
)
def _emb_lookup(table_hbm, idx_hbm, out_hbm, idx_v, rows_v, out_v,
                gsem0, gsem1, osem0, osem1):
    wid = lax.axis_index("s") * NC + lax.axis_index("c")
    gsems = (gsem0, gsem1)
    osems = (osem0, osem1)

    pltpu.sync_copy(idx_hbm.at[pl.ds(wid * IDX_RPW, IDX_RPW)], idx_v)

    def fire(c):
        buf = c % NBUF
        return [
            pltpu.async_copy(
                table_hbm.at[idx_v.at[c * G + g]],
                rows_v.at[buf].at[pl.ds(g * GW, GW)],
                gsems[buf],
            )
            for g in range(G)
        ]

    def accumulate(buf):
        def tok_body(t, carry):
            r = t * K
            for h in range(DIM // 16):
                d = pl.ds(h * 16, 16)
                out_v[buf, t, d] = (
                    rows_v[buf, r, d]
                    + rows_v[buf, r + 1, d]
                    + rows_v[buf, r + 2, d]
                    + rows_v[buf, r + 3, d]
                )
            return carry

        lax.fori_loop(0, T, tok_body, 0)

    tok0 = wid * TPW
    out_copies = [None] * NBUF
    gather_copies = fire(0)
    for c in range(NCHUNK):
        buf = c % NBUF
        if c + 1 < NCHUNK:
            next_copies = fire(c + 1)
        for cp in gather_copies:
            cp.wait()
        if c + 1 < NCHUNK:
            gather_copies = next_copies
        if out_copies[buf] is not None:
            out_copies[buf].wait()
        accumulate(buf)
        out_copies[buf] = pltpu.async_copy(
            out_v.at[buf],
            out_hbm.at[pl.ds(tok0 + c * T, T)],
            osems[buf],
        )
    for cp in out_copies:
        if cp is not None:
            cp.wait()


def kernel(x, table):
    idx = x.reshape(NTOK * K).astype(jnp.int32).reshape(IDX_ROWS, GW)
    out = _emb_lookup(table, idx)
    return out.reshape(B, L, DIM)


# speedup vs baseline: 1.7833x; 1.0021x over previous
import functools

import jax
import jax.numpy as jnp
from jax import lax
from jax.experimental import pallas as pl
from jax.experimental.pallas import tpu as pltpu
from jax.experimental.pallas import tpu_sc as plsc

VOCAB = 1000000
DIM = 32
B = 4096
L = 50
K = 4

NC = 2
NS = 16
NW = NC * NS
NTOK = B * L
TPW = NTOK // NW
T = 256
NCHUNK = TPW // T
RPC = T * K
GW = 128
G = RPC // GW
IDX_ROWS = NTOK * K // GW
IDX_RPW = IDX_ROWS // NW
NBUF = 2

_mesh = plsc.VectorSubcoreMesh(core_axis_name="c", subcore_axis_name="s")


@functools.partial(
    pl.kernel,
    mesh=_mesh,
    out_type=jax.ShapeDtypeStruct((NTOK, DIM), jnp.float32),
    compiler_params=pltpu.CompilerParams(use_tc_tiling_on_sc=False),
    scratch_types=[
        pltpu.VMEM((NCHUNK, RPC), jnp.int32),
        pltpu.VMEM((NBUF, RPC, DIM), jnp.float32),
        pltpu.VMEM((NBUF, T, DIM), jnp.float32),
        pltpu.SemaphoreType.DMA,
        pltpu.SemaphoreType.DMA,
        pltpu.SemaphoreType.DMA,
        pltpu.SemaphoreType.DMA,
    ],
)
def _emb_lookup(table_hbm, idx_hbm, out_hbm, idx_v, rows_v, out_v,
                gsem0, gsem1, osem0, osem1):
    wid = lax.axis_index("s") * NC + lax.axis_index("c")
    gsems = (gsem0, gsem1)
    osems = (osem0, osem1)

    pltpu.sync_copy(idx_hbm.at[pl.ds(wid * NCHUNK, NCHUNK)], idx_v)

    def fire(c):
        buf = c % NBUF
        return pltpu.async_copy(
            table_hbm.at[idx_v.at[c]], rows_v.at[buf], gsems[buf]
        )

    def accumulate(buf):
        def tok_body(t, carry):
            r = t * K
            for h in range(DIM // 16):
                d = pl.ds(h * 16, 16)
                out_v[buf, t, d] = (
                    rows_v[buf, r, d]
                    + rows_v[buf, r + 1, d]
                    + rows_v[buf, r + 2, d]
                    + rows_v[buf, r + 3, d]
                )
            return carry

        lax.fori_loop(0, T, tok_body, 0)

    tok0 = wid * TPW
    out_copies = [None] * NBUF
    gather_copy = fire(0)
    for c in range(NCHUNK):
        buf = c % NBUF
        if c + 1 < NCHUNK:
            next_copy = fire(c + 1)
        gather_copy.wait()
        if c + 1 < NCHUNK:
            gather_copy = next_copy
        if out_copies[buf] is not None:
            out_copies[buf].wait()
        accumulate(buf)
        out_copies[buf] = pltpu.async_copy(
            out_v.at[buf],
            out_hbm.at[pl.ds(tok0 + c * T, T)],
            osems[buf],
        )
    for cp in out_copies:
        if cp is not None:
            cp.wait()


def kernel(x, table):
    idx = x.reshape(NTOK * K).astype(jnp.int32).reshape(NW * NCHUNK, RPC)
    out = _emb_lookup(table, idx)
    return out.reshape(B, L, DIM)
